# baseline (speedup 1.0000x reference)
import functools

import jax
import jax.numpy as jnp
from jax import lax
from jax.experimental import pallas as pl
from jax.experimental.pallas import tpu as pltpu
from jax.experimental.pallas import tpu_sc as plsc

N = 10000
E = 160000
D_IN = 256
H = 256
HH = 128
D_OUT = 128

RB = 1000
NBLK = N // RB
C = 128
NTILES = 16
STRIPE = 632
STRIPE_LAST = N - (NTILES - 1) * STRIPE
NCHUNKS = E // C

_PREC = jax.lax.Precision.DEFAULT


@functools.cache
def _make_sc_agg():
    mesh = plsc.VectorSubcoreMesh(core_axis_name="c", subcore_axis_name="s")
    return functools.partial(
        pl.kernel,
        out_type=jax.ShapeDtypeStruct((2 * N, HH), jnp.float32),
        mesh=mesh,
        scratch_types=(
            [pltpu.VMEM((2, C), jnp.int32)] * 3
            + [pltpu.VMEM((C, HH), jnp.float32)] * 3
            + [pltpu.VMEM_SHARED((N, HH), jnp.float32)]
            + [pltpu.SemaphoreType.DMA] * 6
        ),
    )(_sc_agg_body)


def _sc_agg(hflat, edges3, zrows):
    return _make_sc_agg()(hflat, edges3, zrows)


@functools.cache
def _make_sc_cnt():
    mesh = plsc.VectorSubcoreMesh(core_axis_name="c", subcore_axis_name="s")
    return functools.partial(
        pl.kernel,
        out_type=jax.ShapeDtypeStruct((2 * N, HH), jnp.float32),
        mesh=mesh,
        scratch_types=(
            [pltpu.VMEM((C,), jnp.int32)] * 3
            + [pltpu.VMEM((C, HH), jnp.float32)]
            + [pltpu.VMEM_SHARED((N, HH), jnp.float32)]
            + [pltpu.SemaphoreType.DMA] * 3
        ),
    )(_sc_cnt_body)


def _sc_cnt(dst, zrows, ones_in):
    return _make_sc_cnt()(dst, zrows, ones_in)


_CHALF = NCHUNKS // 2
_CPER = _CHALF // NTILES
_CTRI = _CPER // 3


def _sc_cnt_body(dst, zrows, ones_in, cnt_out,
                 dstb0, dstb1, dstb2, buf, cnt_sh, ss0, ss1, ss2):
    c = lax.axis_index("c")
    s = lax.axis_index("s")
    r0 = s * STRIPE

    pltpu.sync_copy(zrows, buf)

    def _zinit(stripe_rows):
        for off, sz in _stripe_chunks(stripe_rows):
            pltpu.sync_copy(buf.at[pl.ds(0, sz)],
                            cnt_sh.at[pl.ds(r0 + off, sz)])

    @pl.when(s < NTILES - 1)
    def _():
        _zinit(STRIPE)

    @pl.when(s == NTILES - 1)
    def _():
        _zinit(STRIPE_LAST)

    pltpu.sync_copy(ones_in, buf)
    plsc.subcore_barrier()

    start = c * _CHALF + s * _CPER + jnp.where(s > 0, 1, 0)
    bufs = ((dstb0, ss0), (dstb1, ss1), (dstb2, ss2))

    def body(i, carry):
        base = (start + 3 * i) * C
        scps = []
        for k, (db, ss) in enumerate(bufs):
            pltpu.sync_copy(dst.at[pl.ds(base + k * C, C)], db)
            scps.append(pltpu.async_copy(buf, cnt_sh.at[db], ss, add=True))
        for scp in scps:
            scp.wait()
        return carry

    lax.fori_loop(0, _CTRI, body, 0)

    @pl.when(s == 0)
    def _():
        base = (c * _CHALF + 3 * _CTRI) * C
        pltpu.sync_copy(dst.at[pl.ds(base, C)], dstb0)
        pltpu.sync_copy(buf, cnt_sh.at[dstb0], add=True)

    plsc.subcore_barrier()

    def _writeout(stripe_rows):
        for off, sz in _stripe_chunks(stripe_rows):
            pltpu.sync_copy(cnt_sh.at[pl.ds(r0 + off, sz)],
                            buf.at[pl.ds(0, sz)])
            pltpu.sync_copy(buf.at[pl.ds(0, sz)],
                            cnt_out.at[pl.ds(c * N + r0 + off, sz)])

    @pl.when(s < NTILES - 1)
    def _():
        _writeout(STRIPE)

    @pl.when(s == NTILES - 1)
    def _():
        _writeout(STRIPE_LAST)


def _stripe_chunks(stripe_rows):
    full, tail = divmod(stripe_rows, C)
    sizes = [C] * full + ([tail] if tail else [])
    offs = [k * C for k in range(len(sizes))]
    return list(zip(offs, sizes))


_PER = NCHUNKS // NTILES
_LAST = NCHUNKS - (NTILES - 1) * _PER
_TRI = _PER // 3
_TRI_LAST = _LAST // 3
_TAIL_LAST = _LAST - 3 * _TRI_LAST


def _sc_agg_body(hflat, edges3, zrows, agg_out,
                 eb0, eb1, eb2, rows0, rows1, rows2, agg_sh,
                 gs0, gs1, gs2, ss0, ss1, ss2):
    c = lax.axis_index("c")
    s = lax.axis_index("s")
    coff = c * N
    r0 = s * STRIPE

    pltpu.sync_copy(zrows, rows0)

    def _zinit(stripe_rows):
        for off, sz in _stripe_chunks(stripe_rows):
            pltpu.sync_copy(rows0.at[pl.ds(0, sz)],
                            agg_sh.at[pl.ds(r0 + off, sz)])

    @pl.when(s < NTILES - 1)
    def _():
        _zinit(STRIPE)

    @pl.when(s == NTILES - 1)
    def _():
        _zinit(STRIPE_LAST)

    plsc.subcore_barrier()

    start = s * _PER
    bufs = ((eb0, rows0, gs0, ss0),
            (eb1, rows1, gs1, ss1),
            (eb2, rows2, gs2, ss2))

    def _fetch(chunk, eb, rows, sem):
        pltpu.sync_copy(edges3.at[chunk], eb)
        for t in range(C // 16):
            sl = pl.ds(t * 16, 16)
            eb[0, sl] = eb[0, sl] + coff
        cp = pltpu.make_async_copy(hflat.at[eb.at[0]], rows, sem)
        cp.start()
        return cp

    for k, (eb, rw, gs, _) in enumerate(bufs):
        _fetch(start + k, eb, rw, gs)

    def body(i, carry):
        for eb, rw, gs, ss in bufs:
            pltpu.make_async_copy(hflat.at[eb.at[0]], rw, gs).wait()
            pltpu.async_copy(rw, agg_sh.at[eb.at[1]], ss, add=True)
        nxt = jnp.where(i + 1 < _TRI, start + 3 * (i + 1), start)
        for k, (eb, rw, gs, ss) in enumerate(bufs):
            pltpu.make_async_copy(rw, agg_sh.at[eb.at[1]], ss).wait()
            _fetch(nxt + k, eb, rw, gs)
        return carry

    lax.fori_loop(0, _TRI, body, 0)

    for eb, rw, gs, _ in bufs:
        pltpu.make_async_copy(hflat.at[eb.at[0]], rw, gs).wait()

    @pl.when(s == NTILES - 1)
    def _():
        tail0 = (NTILES - 1) * _PER + 3 * _TRI_LAST
        for k in range(_TAIL_LAST):
            eb, rw, gs, _ = bufs[k]
            _fetch(tail0 + k, eb, rw, gs).wait()
            pltpu.sync_copy(rw, agg_sh.at[eb.at[1]], add=True)

    plsc.subcore_barrier()

    def _writeout(stripe_rows):
        for off, sz in _stripe_chunks(stripe_rows):
            pltpu.sync_copy(agg_sh.at[pl.ds(r0 + off, sz)],
                            rows0.at[pl.ds(0, sz)])
            pltpu.sync_copy(rows0.at[pl.ds(0, sz)],
                            agg_out.at[pl.ds(coff + r0 + off, sz)])

    @pl.when(s < NTILES - 1)
    def _():
        _writeout(STRIPE)

    @pl.when(s == NTILES - 1)
    def _():
        _writeout(STRIPE_LAST)


def _h0_body(x_ref, wt_ref, b_ref, o_ref):
    o_ref[...] = (
        jnp.dot(x_ref[...], wt_ref[...],
                preferred_element_type=jnp.float32, precision=_PREC)
        + b_ref[...]
    )


def _h0(x, WT, b):
    return pl.pallas_call(
        _h0_body,
        grid=(2, NBLK),
        in_specs=[
            pl.BlockSpec((RB, D_IN), lambda h, i: (i, 0)),
            pl.BlockSpec((D_IN, HH), lambda h, i: (0, h)),
            pl.BlockSpec((1, HH), lambda h, i: (0, h)),
        ],
        out_specs=pl.BlockSpec((RB, HH), lambda h, i: (h * NBLK + i, 0)),
        out_shape=jax.ShapeDtypeStruct((2 * N, HH), jnp.float32),
    )(x, WT, b)


def _linout(agg_lo, agg_hi, cnt0, cnt1, h_lo, h_hi, wlt, wrt, b):
    inv = 1.0 / jnp.maximum(cnt0[:, 0:1] + cnt1[:, 0:1], 1.0)
    return (
        jnp.dot(agg_lo[...] * inv, wlt[0:HH, :],
                preferred_element_type=jnp.float32, precision=_PREC)
        + jnp.dot(agg_hi[...] * inv, wlt[HH:, :],
                  preferred_element_type=jnp.float32, precision=_PREC)
        + jnp.dot(h_lo[...], wrt[0:HH, :],
                  preferred_element_type=jnp.float32, precision=_PREC)
        + jnp.dot(h_hi[...], wrt[HH:, :],
                  preferred_element_type=jnp.float32, precision=_PREC)
        + b[...]
    )


def _accum_stats(out, i, sum_ref, sq_ref):
    @pl.when(i == 0)
    def _():
        sum_ref[...] = jnp.zeros_like(sum_ref)
        sq_ref[...] = jnp.zeros_like(sq_ref)

    sum_ref[...] += jnp.sum(out, axis=0, keepdims=True)
    sq_ref[...] += jnp.sum(out * out, axis=0, keepdims=True)


def _bn_apply(oscr, i, sum_ref, sq_ref, g_ref, be_ref):
    mean = sum_ref[...] * (1.0 / N)
    var = sq_ref[...] * (1.0 / N) - mean * mean
    alpha = g_ref[...] * lax.rsqrt(var + 1e-5)
    shift = be_ref[...] - mean * alpha
    blk = oscr[pl.ds(i * RB, RB), :]
    return jnp.maximum(blk * alpha + shift, 0.0)


def _lin_apply_body(agg_lo, agg_hi, cnt0, cnt1, h_lo, h_hi, wlt, wrt, b,
                    g_ref, be_ref, o_ref, oscr, sum_ref, sq_ref):
    p = pl.program_id(0)
    i = pl.program_id(1)

    @pl.when(p == 0)
    def _():
        out = _linout(agg_lo, agg_hi, cnt0, cnt1, h_lo, h_hi, wlt, wrt, b)
        oscr[pl.ds(i * RB, RB), :] = out
        _accum_stats(out, i, sum_ref, sq_ref)

    @pl.when(p > 0)
    def _():
        v = _bn_apply(oscr, i, sum_ref, sq_ref, g_ref, be_ref)

        @pl.when(p == 1)
        def _():
            o_ref[...] = h_lo[...] + v[:, 0:HH]

        @pl.when(p == 2)
        def _():
            o_ref[...] = h_hi[...] + v[:, HH:]


def _k_lin_apply(agg, cnt, h, WlT, WrT, b, g, be):
    return pl.pallas_call(
        _lin_apply_body,
        grid=(3, NBLK),
        in_specs=[
            pl.BlockSpec((RB, HH), lambda p, i: (i, 0)),
            pl.BlockSpec((RB, HH), lambda p, i: (NBLK + i, 0)),
            pl.BlockSpec((RB, HH), lambda p, i: (i, 0)),
            pl.BlockSpec((RB, HH), lambda p, i: (NBLK + i, 0)),
            pl.BlockSpec((RB, HH), lambda p, i: (i, 0)),
            pl.BlockSpec((RB, HH), lambda p, i: (NBLK + i, 0)),
            pl.BlockSpec((H, H), lambda p, i: (0, 0)),
            pl.BlockSpec((H, H), lambda p, i: (0, 0)),
            pl.BlockSpec((1, H), lambda p, i: (0, 0)),
            pl.BlockSpec((1, H), lambda p, i: (0, 0)),
            pl.BlockSpec((1, H), lambda p, i: (0, 0)),
        ],
        out_specs=pl.BlockSpec(
            (RB, HH),
            lambda p, i: (jnp.where(p == 0, 0, jnp.where(p == 1, i, NBLK + i)),
                          0)),
        out_shape=jax.ShapeDtypeStruct((2 * N, HH), jnp.float32),
        scratch_shapes=[
            pltpu.VMEM((N, H), jnp.float32),
            pltpu.VMEM((1, H), jnp.float32),
            pltpu.VMEM((1, H), jnp.float32),
        ],
    )(agg, agg, cnt, cnt, h, h, WlT, WrT, b, g, be)


def _lin_apply_dec_body(agg_lo, agg_hi, cnt0, cnt1, h_lo, h_hi, wlt, wrt, b,
                        g_ref, be_ref, wdt, bd, o_ref, oscr, sum_ref, sq_ref):
    p = pl.program_id(0)
    i = pl.program_id(1)

    @pl.when(p == 0)
    def _():
        out = _linout(agg_lo, agg_hi, cnt0, cnt1, h_lo, h_hi, wlt, wrt, b)
        oscr[pl.ds(i * RB, RB), :] = out
        _accum_stats(out, i, sum_ref, sq_ref)

    @pl.when(p == 1)
    def _():
        v = _bn_apply(oscr, i, sum_ref, sq_ref, g_ref, be_ref)
        hn_lo = h_lo[...] + v[:, 0:HH]
        hn_hi = h_hi[...] + v[:, HH:]
        o_ref[...] = (
            jnp.dot(hn_lo, wdt[0:HH, :],
                    preferred_element_type=jnp.float32, precision=_PREC)
            + jnp.dot(hn_hi, wdt[HH:, :],
                      preferred_element_type=jnp.float32, precision=_PREC)
            + bd[...]
        )


def _k_lin_apply_dec(agg, cnt, h, WlT, WrT, b, g, be, WdT, bd):
    return pl.pallas_call(
        _lin_apply_dec_body,
        grid=(2, NBLK),
        in_specs=[
            pl.BlockSpec((RB, HH), lambda p, i: (i, 0)),
            pl.BlockSpec((RB, HH), lambda p, i: (NBLK + i, 0)),
            pl.BlockSpec((RB, HH), lambda p, i: (i, 0)),
            pl.BlockSpec((RB, HH), lambda p, i: (NBLK + i, 0)),
            pl.BlockSpec((RB, HH), lambda p, i: (i, 0)),
            pl.BlockSpec((RB, HH), lambda p, i: (NBLK + i, 0)),
            pl.BlockSpec((H, H), lambda p, i: (0, 0)),
            pl.BlockSpec((H, H), lambda p, i: (0, 0)),
            pl.BlockSpec((1, H), lambda p, i: (0, 0)),
            pl.BlockSpec((1, H), lambda p, i: (0, 0)),
            pl.BlockSpec((1, H), lambda p, i: (0, 0)),
            pl.BlockSpec((H, D_OUT), lambda p, i: (0, 0)),
            pl.BlockSpec((1, D_OUT), lambda p, i: (0, 0)),
        ],
        out_specs=pl.BlockSpec(
            (RB, D_OUT), lambda p, i: (jnp.where(p == 0, 0, i), 0)),
        out_shape=jax.ShapeDtypeStruct((N, D_OUT), jnp.float32),
        scratch_shapes=[
            pltpu.VMEM((N, H), jnp.float32),
            pltpu.VMEM((1, H), jnp.float32),
            pltpu.VMEM((1, H), jnp.float32),
        ],
    )(agg, agg, cnt, cnt, h, h, WlT, WrT, b, g, be, WdT, bd)


def kernel(x, edge_index, W_in, b_in, Wl0, Wr0, bl0, g0, be0,
           Wl1, Wr1, bl1, g1, be1, W_dec, b_dec):
    dst = edge_index[1]
    edges3 = jnp.transpose(edge_index.reshape(2, NCHUNKS, C), (1, 0, 2))
    zrows = jnp.zeros((C, HH), jnp.float32)
    ones_in = jnp.ones((C, HH), jnp.float32)

    cnt = _sc_cnt(dst, zrows, ones_in)
    h = _h0(x, W_in.T, b_in.reshape(1, H))

    agg = _sc_agg(h, edges3, zrows)
    h = _k_lin_apply(agg, cnt, h, Wl0.T, Wr0.T, bl0.reshape(1, H),
                     g0.reshape(1, H), be0.reshape(1, H))

    agg = _sc_agg(h, edges3, zrows)
    return _k_lin_apply_dec(agg, cnt, h, Wl1.T, Wr1.T, bl1.reshape(1, H),
                            g1.reshape(1, H), be1.reshape(1, H),
                            W_dec.T, b_dec.reshape(1, D_OUT))


# speedup vs baseline: 1.0257x; 1.0257x over previous
import functools

import jax
import jax.numpy as jnp
from jax import lax
from jax.experimental import pallas as pl
from jax.experimental.pallas import tpu as pltpu
from jax.experimental.pallas import tpu_sc as plsc

N = 10000
E = 160000
D_IN = 256
H = 256
HH = 128
D_OUT = 128

RB = 1000
NBLK = N // RB
C = 128
NTILES = 16
STRIPE = 632
STRIPE_LAST = N - (NTILES - 1) * STRIPE
NCHUNKS = E // C

_PREC = jax.lax.Precision.DEFAULT


@functools.cache
def _make_sc_agg():
    mesh = plsc.VectorSubcoreMesh(core_axis_name="c", subcore_axis_name="s")
    return functools.partial(
        pl.kernel,
        out_type=jax.ShapeDtypeStruct((2 * N, HH), jnp.float32),
        mesh=mesh,
        scratch_types=(
            [pltpu.VMEM((2, C), jnp.int32)] * 3
            + [pltpu.VMEM((C, HH), jnp.float32)] * 3
            + [pltpu.VMEM_SHARED((N, HH), jnp.float32)]
            + [pltpu.SemaphoreType.DMA] * 6
        ),
    )(_sc_agg_body)


def _sc_agg(hflat, edges3, zrows):
    return _make_sc_agg()(hflat, edges3, zrows)


@functools.cache
def _make_sc_cnt():
    mesh = plsc.VectorSubcoreMesh(core_axis_name="c", subcore_axis_name="s")
    return functools.partial(
        pl.kernel,
        out_type=jax.ShapeDtypeStruct((2 * N, HH), jnp.float32),
        mesh=mesh,
        scratch_types=(
            [pltpu.VMEM((C,), jnp.int32)] * 3
            + [pltpu.VMEM((C, HH), jnp.float32)]
            + [pltpu.VMEM_SHARED((N, HH), jnp.float32)]
            + [pltpu.SemaphoreType.DMA] * 3
        ),
    )(_sc_cnt_body)


def _sc_cnt(dst, zrows, ones_in):
    return _make_sc_cnt()(dst, zrows, ones_in)


_CHALF = NCHUNKS // 2
_CPER = _CHALF // NTILES
_CTRI = _CPER // 3


def _sc_cnt_body(dst, zrows, ones_in, cnt_out,
                 dstb0, dstb1, dstb2, buf, cnt_sh, ss0, ss1, ss2):
    c = lax.axis_index("c")
    s = lax.axis_index("s")
    r0 = s * STRIPE

    pltpu.sync_copy(zrows, buf)

    def _zinit(stripe_rows):
        for off, sz in _stripe_chunks(stripe_rows):
            pltpu.sync_copy(buf.at[pl.ds(0, sz)],
                            cnt_sh.at[pl.ds(r0 + off, sz)])

    @pl.when(s < NTILES - 1)
    def _():
        _zinit(STRIPE)

    @pl.when(s == NTILES - 1)
    def _():
        _zinit(STRIPE_LAST)

    pltpu.sync_copy(ones_in, buf)
    plsc.subcore_barrier()

    start = c * _CHALF + s * _CPER + jnp.where(s > 0, 1, 0)
    bufs = ((dstb0, ss0), (dstb1, ss1), (dstb2, ss2))

    def body(i, carry):
        base = (start + 3 * i) * C
        scps = []
        for k, (db, ss) in enumerate(bufs):
            pltpu.sync_copy(dst.at[pl.ds(base + k * C, C)], db)
            scps.append(pltpu.async_copy(buf, cnt_sh.at[db], ss, add=True))
        for scp in scps:
            scp.wait()
        return carry

    lax.fori_loop(0, _CTRI, body, 0)

    @pl.when(s == 0)
    def _():
        base = (c * _CHALF + 3 * _CTRI) * C
        pltpu.sync_copy(dst.at[pl.ds(base, C)], dstb0)
        pltpu.sync_copy(buf, cnt_sh.at[dstb0], add=True)

    plsc.subcore_barrier()

    def _writeout(stripe_rows):
        for off, sz in _stripe_chunks(stripe_rows):
            pltpu.sync_copy(cnt_sh.at[pl.ds(r0 + off, sz)],
                            buf.at[pl.ds(0, sz)])
            pltpu.sync_copy(buf.at[pl.ds(0, sz)],
                            cnt_out.at[pl.ds(c * N + r0 + off, sz)])

    @pl.when(s < NTILES - 1)
    def _():
        _writeout(STRIPE)

    @pl.when(s == NTILES - 1)
    def _():
        _writeout(STRIPE_LAST)


def _stripe_chunks(stripe_rows):
    full, tail = divmod(stripe_rows, C)
    sizes = [C] * full + ([tail] if tail else [])
    offs = [k * C for k in range(len(sizes))]
    return list(zip(offs, sizes))


_PER = NCHUNKS // NTILES
_LAST = NCHUNKS - (NTILES - 1) * _PER
_TRI = _PER // 3
_TRI_LAST = _LAST // 3
_TAIL_LAST = _LAST - 3 * _TRI_LAST


def _sc_agg_body(hflat, edges3, zrows, agg_out,
                 eb0, eb1, eb2, rows0, rows1, rows2, agg_sh,
                 gs0, gs1, gs2, ss0, ss1, ss2):
    c = lax.axis_index("c")
    s = lax.axis_index("s")
    coff = c * N
    r0 = s * STRIPE

    pltpu.sync_copy(zrows, rows0)

    def _zinit(stripe_rows):
        for off, sz in _stripe_chunks(stripe_rows):
            pltpu.sync_copy(rows0.at[pl.ds(0, sz)],
                            agg_sh.at[pl.ds(r0 + off, sz)])

    @pl.when(s < NTILES - 1)
    def _():
        _zinit(STRIPE)

    @pl.when(s == NTILES - 1)
    def _():
        _zinit(STRIPE_LAST)

    plsc.subcore_barrier()

    start = s * _PER
    bufs = ((eb0, rows0, gs0, ss0),
            (eb1, rows1, gs1, ss1),
            (eb2, rows2, gs2, ss2))

    def _fetch(chunk, eb, rows, sem):
        pltpu.sync_copy(edges3.at[chunk], eb)
        for t in range(C // 16):
            sl = pl.ds(t * 16, 16)
            eb[0, sl] = eb[0, sl] + coff
        cp = pltpu.make_async_copy(hflat.at[eb.at[0]], rows, sem)
        cp.start()
        return cp

    for k, (eb, rw, gs, _) in enumerate(bufs):
        _fetch(start + k, eb, rw, gs)

    def body(i, carry):
        for eb, rw, gs, ss in bufs:
            pltpu.make_async_copy(hflat.at[eb.at[0]], rw, gs).wait()
            pltpu.async_copy(rw, agg_sh.at[eb.at[1]], ss, add=True)
        nxt = jnp.where(i + 1 < _TRI, start + 3 * (i + 1), start)
        for k, (eb, rw, gs, ss) in enumerate(bufs):
            pltpu.make_async_copy(rw, agg_sh.at[eb.at[1]], ss).wait()
            _fetch(nxt + k, eb, rw, gs)
        return carry

    lax.fori_loop(0, _TRI, body, 0)

    for eb, rw, gs, _ in bufs:
        pltpu.make_async_copy(hflat.at[eb.at[0]], rw, gs).wait()

    @pl.when(s == NTILES - 1)
    def _():
        tail0 = (NTILES - 1) * _PER + 3 * _TRI_LAST
        for k in range(_TAIL_LAST):
            eb, rw, gs, _ = bufs[k]
            _fetch(tail0 + k, eb, rw, gs).wait()
            pltpu.sync_copy(rw, agg_sh.at[eb.at[1]], add=True)

    plsc.subcore_barrier()

    def _writeout(stripe_rows):
        for off, sz in _stripe_chunks(stripe_rows):
            pltpu.sync_copy(agg_sh.at[pl.ds(r0 + off, sz)],
                            rows0.at[pl.ds(0, sz)])
            pltpu.sync_copy(rows0.at[pl.ds(0, sz)],
                            agg_out.at[pl.ds(coff + r0 + off, sz)])

    @pl.when(s < NTILES - 1)
    def _():
        _writeout(STRIPE)

    @pl.when(s == NTILES - 1)
    def _():
        _writeout(STRIPE_LAST)


def _h0_body(x_ref, wt_ref, b_ref, o_ref):
    o_ref[...] = (
        jnp.dot(x_ref[...], wt_ref[...],
                preferred_element_type=jnp.float32, precision=_PREC)
        + b_ref[...]
    )


def _h0(x, WT, b):
    return pl.pallas_call(
        _h0_body,
        grid=(2, NBLK),
        in_specs=[
            pl.BlockSpec((RB, D_IN), lambda h, i: (i, 0)),
            pl.BlockSpec((D_IN, HH), lambda h, i: (0, h)),
            pl.BlockSpec((1, HH), lambda h, i: (0, h)),
        ],
        out_specs=pl.BlockSpec((RB, HH), lambda h, i: (h * NBLK + i, 0)),
        out_shape=jax.ShapeDtypeStruct((2 * N, HH), jnp.float32),
    )(x, WT, b)


def _linout(agg_lo, agg_hi, cnt0, cnt1, h_lo, h_hi, wlt, wrt, b):
    inv = 1.0 / jnp.maximum(cnt0[:, 0:1] + cnt1[:, 0:1], 1.0)
    return (
        jnp.dot(agg_lo[...] * inv, wlt[0:HH, :],
                preferred_element_type=jnp.float32, precision=_PREC)
        + jnp.dot(agg_hi[...] * inv, wlt[HH:, :],
                  preferred_element_type=jnp.float32, precision=_PREC)
        + jnp.dot(h_lo[...], wrt[0:HH, :],
                  preferred_element_type=jnp.float32, precision=_PREC)
        + jnp.dot(h_hi[...], wrt[HH:, :],
                  preferred_element_type=jnp.float32, precision=_PREC)
        + b[...]
    )


def _accum_stats(out, i, sum_ref, sq_ref):
    @pl.when(i == 0)
    def _():
        sum_ref[...] = jnp.zeros_like(sum_ref)
        sq_ref[...] = jnp.zeros_like(sq_ref)

    sum_ref[...] += jnp.sum(out, axis=0, keepdims=True)
    sq_ref[...] += jnp.sum(out * out, axis=0, keepdims=True)


def _bn_apply(oscr, i, sum_ref, sq_ref, g_ref, be_ref):
    mean = sum_ref[...] * (1.0 / N)
    var = sq_ref[...] * (1.0 / N) - mean * mean
    alpha = g_ref[...] * lax.rsqrt(var + 1e-5)
    shift = be_ref[...] - mean * alpha
    blk = oscr[pl.ds(i * RB, RB), :]
    return jnp.maximum(blk * alpha + shift, 0.0)


def _lin_apply_body(agg_lo, agg_hi, cnt0, cnt1, h_lo, h_hi, wlt, wrt, b,
                    g_ref, be_ref, o_ref, oscr, sum_ref, sq_ref):
    p = pl.program_id(0)
    i = pl.program_id(1)

    @pl.when(p == 0)
    def _():
        out = _linout(agg_lo, agg_hi, cnt0, cnt1, h_lo, h_hi, wlt, wrt, b)
        oscr[pl.ds(i * RB, RB), :] = out
        _accum_stats(out, i, sum_ref, sq_ref)

    @pl.when(p > 0)
    def _():
        v = _bn_apply(oscr, i, sum_ref, sq_ref, g_ref, be_ref)

        @pl.when(p == 1)
        def _():
            o_ref[...] = h_lo[...] + v[:, 0:HH]

        @pl.when(p == 2)
        def _():
            o_ref[...] = h_hi[...] + v[:, HH:]


def _k_lin_apply(agg, cnt, h, WlT, WrT, b, g, be):
    return pl.pallas_call(
        _lin_apply_body,
        grid=(3, NBLK),
        in_specs=[
            pl.BlockSpec((RB, HH), lambda p, i: (jnp.where(p == 0, i, 0), 0)),
            pl.BlockSpec((RB, HH),
                         lambda p, i: (jnp.where(p == 0, NBLK + i, NBLK), 0)),
            pl.BlockSpec((RB, HH), lambda p, i: (jnp.where(p == 0, i, 0), 0)),
            pl.BlockSpec((RB, HH),
                         lambda p, i: (jnp.where(p == 0, NBLK + i, NBLK), 0)),
            pl.BlockSpec((RB, HH), lambda p, i: (i, 0)),
            pl.BlockSpec((RB, HH), lambda p, i: (NBLK + i, 0)),
            pl.BlockSpec((H, H), lambda p, i: (0, 0)),
            pl.BlockSpec((H, H), lambda p, i: (0, 0)),
            pl.BlockSpec((1, H), lambda p, i: (0, 0)),
            pl.BlockSpec((1, H), lambda p, i: (0, 0)),
            pl.BlockSpec((1, H), lambda p, i: (0, 0)),
        ],
        out_specs=pl.BlockSpec(
            (RB, HH),
            lambda p, i: (jnp.where(p == 0, 0, jnp.where(p == 1, i, NBLK + i)),
                          0)),
        out_shape=jax.ShapeDtypeStruct((2 * N, HH), jnp.float32),
        scratch_shapes=[
            pltpu.VMEM((N, H), jnp.float32),
            pltpu.VMEM((1, H), jnp.float32),
            pltpu.VMEM((1, H), jnp.float32),
        ],
    )(agg, agg, cnt, cnt, h, h, WlT, WrT, b, g, be)


def _lin_apply_dec_body(agg_lo, agg_hi, cnt0, cnt1, h_lo, h_hi, wlt, wrt, b,
                        g_ref, be_ref, wdt, bd, o_ref, oscr, sum_ref, sq_ref):
    p = pl.program_id(0)
    i = pl.program_id(1)

    @pl.when(p == 0)
    def _():
        out = _linout(agg_lo, agg_hi, cnt0, cnt1, h_lo, h_hi, wlt, wrt, b)
        oscr[pl.ds(i * RB, RB), :] = out
        _accum_stats(out, i, sum_ref, sq_ref)

    @pl.when(p == 1)
    def _():
        v = _bn_apply(oscr, i, sum_ref, sq_ref, g_ref, be_ref)
        hn_lo = h_lo[...] + v[:, 0:HH]
        hn_hi = h_hi[...] + v[:, HH:]
        o_ref[...] = (
            jnp.dot(hn_lo, wdt[0:HH, :],
                    preferred_element_type=jnp.float32, precision=_PREC)
            + jnp.dot(hn_hi, wdt[HH:, :],
                      preferred_element_type=jnp.float32, precision=_PREC)
            + bd[...]
        )


def _k_lin_apply_dec(agg, cnt, h, WlT, WrT, b, g, be, WdT, bd):
    return pl.pallas_call(
        _lin_apply_dec_body,
        grid=(2, NBLK),
        in_specs=[
            pl.BlockSpec((RB, HH), lambda p, i: (jnp.where(p == 0, i, 0), 0)),
            pl.BlockSpec((RB, HH),
                         lambda p, i: (jnp.where(p == 0, NBLK + i, NBLK), 0)),
            pl.BlockSpec((RB, HH), lambda p, i: (jnp.where(p == 0, i, 0), 0)),
            pl.BlockSpec((RB, HH),
                         lambda p, i: (jnp.where(p == 0, NBLK + i, NBLK), 0)),
            pl.BlockSpec((RB, HH), lambda p, i: (i, 0)),
            pl.BlockSpec((RB, HH), lambda p, i: (NBLK + i, 0)),
            pl.BlockSpec((H, H), lambda p, i: (0, 0)),
            pl.BlockSpec((H, H), lambda p, i: (0, 0)),
            pl.BlockSpec((1, H), lambda p, i: (0, 0)),
            pl.BlockSpec((1, H), lambda p, i: (0, 0)),
            pl.BlockSpec((1, H), lambda p, i: (0, 0)),
            pl.BlockSpec((H, D_OUT), lambda p, i: (0, 0)),
            pl.BlockSpec((1, D_OUT), lambda p, i: (0, 0)),
        ],
        out_specs=pl.BlockSpec(
            (RB, D_OUT), lambda p, i: (jnp.where(p == 0, 0, i), 0)),
        out_shape=jax.ShapeDtypeStruct((N, D_OUT), jnp.float32),
        scratch_shapes=[
            pltpu.VMEM((N, H), jnp.float32),
            pltpu.VMEM((1, H), jnp.float32),
            pltpu.VMEM((1, H), jnp.float32),
        ],
    )(agg, agg, cnt, cnt, h, h, WlT, WrT, b, g, be, WdT, bd)


def kernel(x, edge_index, W_in, b_in, Wl0, Wr0, bl0, g0, be0,
           Wl1, Wr1, bl1, g1, be1, W_dec, b_dec):
    dst = edge_index[1]
    edges3 = jnp.transpose(edge_index.reshape(2, NCHUNKS, C), (1, 0, 2))
    zrows = jnp.zeros((C, HH), jnp.float32)
    ones_in = jnp.ones((C, HH), jnp.float32)

    cnt = _sc_cnt(dst, zrows, ones_in)
    h = _h0(x, W_in.T, b_in.reshape(1, H))

    agg = _sc_agg(h, edges3, zrows)
    h = _k_lin_apply(agg, cnt, h, Wl0.T, Wr0.T, bl0.reshape(1, H),
                     g0.reshape(1, H), be0.reshape(1, H))

    agg = _sc_agg(h, edges3, zrows)
    return _k_lin_apply_dec(agg, cnt, h, Wl1.T, Wr1.T, bl1.reshape(1, H),
                            g1.reshape(1, H), be1.reshape(1, H),
                            W_dec.T, b_dec.reshape(1, D_OUT))
